# baseline (speedup 1.0000x reference)
import functools

import jax
import jax.numpy as jnp
from jax import lax
from jax.experimental import pallas as pl
from jax.experimental.pallas import tpu as pltpu
from jax.experimental.pallas import tpu_sc as plsc

_BATCH = 8192
_F = 768
_LANES = 16

_B_SC = 3072
_B_TC = _BATCH - _B_SC

_NW = 32
_ROWS_PER_W = _B_SC // _NW
_CHUNK = 32
_NCHUNK = _ROWS_PER_W // _CHUNK

_TCB = 512


def _sc_half(x_full, wsub_host):
    mesh = plsc.VectorSubcoreMesh(
        core_axis_name="c", subcore_axis_name="s", num_cores=2, num_subcores=16)

    @functools.partial(
        pl.kernel,
        out_type=jax.ShapeDtypeStruct((_BATCH, _F), jnp.float32),
        mesh=mesh,
        compiler_params=pltpu.CompilerParams(needs_layout_passes=False),
        scratch_types=[
            pltpu.VMEM((_F * 16 // 128, 128), jnp.float32),
            pltpu.VMEM((16, _F), jnp.float32),
            pltpu.VMEM((2, _CHUNK, _F), jnp.float32),
            pltpu.VMEM((2, _CHUNK, _F), jnp.float32),
            pltpu.SemaphoreType.DMA,
            pltpu.SemaphoreType.DMA,
            pltpu.SemaphoreType.DMA,
            pltpu.SemaphoreType.DMA,
        ],
    )
    def run(x_hbm, w_hbm, out_hbm, wsub, tbl, xbuf, obuf,
            sin0, sin1, sout0, sout1):
        wid = lax.axis_index("s") * 2 + lax.axis_index("c")
        row0 = wid * _ROWS_PER_W

        pltpu.sync_copy(w_hbm, wsub)

        lanes = lax.iota(jnp.int32, _LANES)
        for si in range(4):
            def tbody(j, carry, si=si):
                fo = j * _LANES
                flat = (fo + lanes) * 16
                def gath(c):
                    fl = flat + c
                    return plsc.load_gather(
                        wsub, [lax.shift_right_logical(fl, 7),
                               lax.bitwise_and(fl, 127)])
                w0 = gath(4 * si + 0)
                w1 = gath(4 * si + 1)
                w2 = gath(4 * si + 2)
                w3 = gath(4 * si + 3)
                tbl[4 * si + 0, pl.ds(fo, _LANES)] = (-w0 + 9.0 * w1 + 9.0 * w2 - w3) * (1.0 / 16.0)
                tbl[4 * si + 1, pl.ds(fo, _LANES)] = (w0 - 27.0 * w1 + 27.0 * w2 - w3) * (1.0 / 16.0)
                tbl[4 * si + 2, pl.ds(fo, _LANES)] = (w0 - w1 - w2 + w3) * (9.0 / 16.0)
                tbl[4 * si + 3, pl.ds(fo, _LANES)] = (-w0 + 3.0 * w1 - 3.0 * w2 + w3) * (9.0 / 16.0)
                return carry
            lax.fori_loop(0, _F // _LANES, tbody, 0)

        sin = (sin0, sin1)
        sout = (sout0, sout1)

        def in_copy(c):
            return pltpu.make_async_copy(
                x_hbm.at[pl.ds(row0 + c * _CHUNK, _CHUNK), :],
                xbuf.at[c % 2], sin[c % 2])

        def out_copy(c):
            return pltpu.make_async_copy(
                obuf.at[c % 2],
                out_hbm.at[pl.ds(row0 + c * _CHUNK, _CHUNK), :], sout[c % 2])

        in_copy(0).start()
        in_copy(1).start()

        for c in range(_NCHUNK):
            p = c % 2
            in_copy(c).wait()
            if c >= 2:
                out_copy(c - 2).wait()

            def rbody(rr, carry, p=p):
                ra = rr * 2
                rb = ra + 1

                def segbase(r):
                    xv0 = xbuf[p, r, pl.ds(0, _LANES)]
                    t0 = xv0[0] * 4.0 + 4.0
                    sr = lax.convert_element_type(t0, jnp.int32)
                    sf = lax.convert_element_type(sr, jnp.float32)
                    seg = sr - lax.select(sf > t0, 1, 0)
                    seg = lax.max(lax.min(seg, 7), 4)
                    return (seg - 4) * 4

                tba = segbase(ra)
                tbb = segbase(rb)

                @plsc.parallel_loop(0, _F // _LANES, unroll=4)
                def fbody(j, p=p):
                    fo = j * _LANES
                    for r, tb in ((ra, tba), (rb, tbb)):
                        xv = xbuf[p, r, pl.ds(fo, _LANES)]
                        t = xv * 4.0 + 4.0
                        idf = lax.convert_element_type(
                            lax.convert_element_type(t, jnp.int32), jnp.float32)
                        xin = 2.0 * (t - idf) - 1.0
                        acc = tbl[tb + 3, pl.ds(fo, _LANES)]
                        acc = acc * xin + tbl[tb + 2, pl.ds(fo, _LANES)]
                        acc = acc * xin + tbl[tb + 1, pl.ds(fo, _LANES)]
                        acc = acc * xin + tbl[tb + 0, pl.ds(fo, _LANES)]
                        obuf[p, r, pl.ds(fo, _LANES)] = acc

                return carry

            lax.fori_loop(0, _CHUNK // 2, rbody, 0)
            out_copy(c).start()
            if c + 2 < _NCHUNK:
                in_copy(c + 2).start()

        out_copy(_NCHUNK - 2).wait()
        out_copy(_NCHUNK - 1).wait()

    return run(x_full, wsub_host)


def _tc_body(wt_ref, x_ref, o_ref):
    xb = x_ref[...]
    t = xb * 4.0 + 4.0
    idf = lax.convert_element_type(
        lax.convert_element_type(t, jnp.int32), jnp.float32)
    xin = 2.0 * (t - idf) - 1.0
    seg_f = jnp.clip(idf[:, 0:1], 4.0, 7.0)
    acc = jnp.zeros_like(xb)
    for si in range(4):
        w0 = wt_ref[4 * si + 0:4 * si + 1, :]
        w1 = wt_ref[4 * si + 1:4 * si + 2, :]
        w2 = wt_ref[4 * si + 2:4 * si + 3, :]
        w3 = wt_ref[4 * si + 3:4 * si + 4, :]
        c0 = (-w0 + 9.0 * w1 + 9.0 * w2 - w3) * (1.0 / 16.0)
        c1 = (w0 - 27.0 * w1 + 27.0 * w2 - w3) * (1.0 / 16.0)
        c2 = (w0 - w1 - w2 + w3) * (9.0 / 16.0)
        c3 = (-w0 + 3.0 * w1 - 3.0 * w2 + w3) * (9.0 / 16.0)
        h = ((c3 * xin + c2) * xin + c1) * xin + c0
        acc = acc + jnp.where(seg_f == float(si + 4), h, 0.0)
    o_ref[...] = acc


def _tc_half(x, wt_host):
    off = _B_SC // _TCB
    return pl.pallas_call(
        _tc_body,
        out_shape=jax.ShapeDtypeStruct((_B_TC, _F), jnp.float32),
        grid=(_B_TC // _TCB,),
        in_specs=[
            pl.BlockSpec((16, _F), lambda i: (0, 0)),
            pl.BlockSpec((_TCB, _F), lambda i: (i + off, 0)),
        ],
        out_specs=pl.BlockSpec((_TCB, _F), lambda i: (i, 0)),
    )(wt_host, x)


def _assemble_body(t_ref, full_ref, o_ref):
    del full_ref
    o_ref[...] = t_ref[...]


def _assemble(out_tc, full):
    off = _B_SC // _TCB
    return pl.pallas_call(
        _assemble_body,
        out_shape=jax.ShapeDtypeStruct((_BATCH, _F), jnp.float32),
        grid=(_B_TC // _TCB,),
        in_specs=[
            pl.BlockSpec((_TCB, _F), lambda i: (i, 0)),
            pl.BlockSpec(memory_space=pl.ANY),
        ],
        out_specs=pl.BlockSpec((_TCB, _F), lambda i: (i + off, 0)),
        input_output_aliases={1: 0},
    )(out_tc, full)


def kernel(x, w):
    wsub_host = lax.slice(w, (0, 16), (_F, 32))
    wt_host = wsub_host.T
    wflat_host = wsub_host.reshape(_F * 16 // 128, 128)
    full = _sc_half(x, wflat_host)
    out_tc = _tc_half(x, wt_host)
    return _assemble(out_tc, full)


# speedup vs baseline: 1.0585x; 1.0585x over previous
import functools

import jax
import jax.numpy as jnp
from jax import lax
from jax.experimental import pallas as pl
from jax.experimental.pallas import tpu as pltpu
from jax.experimental.pallas import tpu_sc as plsc

_BATCH = 8192
_F = 768
_LANES = 16

_B_SC = 3072
_B_TC = _BATCH - _B_SC

_NW = 32
_ROWS_PER_W = _B_SC // _NW
_CHUNK = 32
_NCHUNK = _ROWS_PER_W // _CHUNK

_TCB = 512


def _sc_half(x_full, wsub_host):
    mesh = plsc.VectorSubcoreMesh(
        core_axis_name="c", subcore_axis_name="s", num_cores=2, num_subcores=16)

    @functools.partial(
        pl.kernel,
        out_type=jax.ShapeDtypeStruct((_BATCH, _F), jnp.float32),
        mesh=mesh,
        compiler_params=pltpu.CompilerParams(needs_layout_passes=False),
        scratch_types=[
            pltpu.VMEM((_F * 16 // 128, 128), jnp.float32),
            pltpu.VMEM((16, _F), jnp.float32),
            pltpu.VMEM((2, _CHUNK, _F), jnp.float32),
            pltpu.VMEM((2, _CHUNK, _F), jnp.float32),
            pltpu.SemaphoreType.DMA,
            pltpu.SemaphoreType.DMA,
            pltpu.SemaphoreType.DMA,
            pltpu.SemaphoreType.DMA,
        ],
    )
    def run(x_hbm, w_hbm, out_hbm, wsub, tbl, xbuf, obuf,
            sin0, sin1, sout0, sout1):
        wid = lax.axis_index("s") * 2 + lax.axis_index("c")
        row0 = wid * _ROWS_PER_W

        pltpu.sync_copy(w_hbm, wsub)

        lanes = lax.iota(jnp.int32, _LANES)
        for si in range(4):
            def tbody(j, carry, si=si):
                fo = j * _LANES
                flat = (fo + lanes) * 16
                def gath(c):
                    fl = flat + c
                    return plsc.load_gather(
                        wsub, [lax.shift_right_logical(fl, 7),
                               lax.bitwise_and(fl, 127)])
                w0 = gath(4 * si + 0)
                w1 = gath(4 * si + 1)
                w2 = gath(4 * si + 2)
                w3 = gath(4 * si + 3)
                tbl[4 * si + 0, pl.ds(fo, _LANES)] = (-w0 + 9.0 * w1 + 9.0 * w2 - w3) * (1.0 / 16.0)
                tbl[4 * si + 1, pl.ds(fo, _LANES)] = (w0 - 27.0 * w1 + 27.0 * w2 - w3) * (1.0 / 16.0)
                tbl[4 * si + 2, pl.ds(fo, _LANES)] = (w0 - w1 - w2 + w3) * (9.0 / 16.0)
                tbl[4 * si + 3, pl.ds(fo, _LANES)] = (-w0 + 3.0 * w1 - 3.0 * w2 + w3) * (9.0 / 16.0)
                return carry
            lax.fori_loop(0, _F // _LANES, tbody, 0)

        sin = (sin0, sin1)
        sout = (sout0, sout1)

        def in_copy(c):
            return pltpu.make_async_copy(
                x_hbm.at[pl.ds(row0 + c * _CHUNK, _CHUNK), :],
                xbuf.at[c % 2], sin[c % 2])

        def out_copy(c):
            return pltpu.make_async_copy(
                obuf.at[c % 2],
                out_hbm.at[pl.ds(row0 + c * _CHUNK, _CHUNK), :], sout[c % 2])

        in_copy(0).start()
        in_copy(1).start()

        for c in range(_NCHUNK):
            p = c % 2
            in_copy(c).wait()
            if c >= 2:
                out_copy(c - 2).wait()

            def rbody(r, carry, p=p):
                xv0 = xbuf[p, r, pl.ds(0, _LANES)]
                t0 = xv0[0] * 4.0 + 4.0
                sr = lax.convert_element_type(t0, jnp.int32)
                sf = lax.convert_element_type(sr, jnp.float32)
                seg = sr - lax.select(sf > t0, 1, 0)
                seg = lax.max(lax.min(seg, 7), 4)
                tb = (seg - 4) * 4

                @plsc.parallel_loop(0, _F // _LANES, unroll=8)
                def fbody(j, p=p):
                    fo = j * _LANES
                    xv = xbuf[p, r, pl.ds(fo, _LANES)]
                    t = xv * 4.0 + 4.0
                    idf = lax.convert_element_type(
                        lax.convert_element_type(t, jnp.int32), jnp.float32)
                    xin = 2.0 * (t - idf) - 1.0
                    acc = tbl[tb + 3, pl.ds(fo, _LANES)]
                    acc = acc * xin + tbl[tb + 2, pl.ds(fo, _LANES)]
                    acc = acc * xin + tbl[tb + 1, pl.ds(fo, _LANES)]
                    acc = acc * xin + tbl[tb + 0, pl.ds(fo, _LANES)]
                    obuf[p, r, pl.ds(fo, _LANES)] = acc

                return carry

            lax.fori_loop(0, _CHUNK, rbody, 0)
            out_copy(c).start()
            if c + 2 < _NCHUNK:
                in_copy(c + 2).start()

        out_copy(_NCHUNK - 2).wait()
        out_copy(_NCHUNK - 1).wait()

    return run(x_full, wsub_host)


def _tc_body(wt_ref, x_ref, o_ref):
    xb = x_ref[...]
    t = xb * 4.0 + 4.0
    idf = lax.convert_element_type(
        lax.convert_element_type(t, jnp.int32), jnp.float32)
    xin = 2.0 * (t - idf) - 1.0
    seg_f = jnp.clip(idf[:, 0:1], 4.0, 7.0)
    acc = jnp.zeros_like(xb)
    for si in range(4):
        w0 = wt_ref[4 * si + 0:4 * si + 1, :]
        w1 = wt_ref[4 * si + 1:4 * si + 2, :]
        w2 = wt_ref[4 * si + 2:4 * si + 3, :]
        w3 = wt_ref[4 * si + 3:4 * si + 4, :]
        c0 = (-w0 + 9.0 * w1 + 9.0 * w2 - w3) * (1.0 / 16.0)
        c1 = (w0 - 27.0 * w1 + 27.0 * w2 - w3) * (1.0 / 16.0)
        c2 = (w0 - w1 - w2 + w3) * (9.0 / 16.0)
        c3 = (-w0 + 3.0 * w1 - 3.0 * w2 + w3) * (9.0 / 16.0)
        h = ((c3 * xin + c2) * xin + c1) * xin + c0
        acc = acc + jnp.where(seg_f == float(si + 4), h, 0.0)
    o_ref[...] = acc


def _tc_half(x, wt_host):
    off = _B_SC // _TCB
    return pl.pallas_call(
        _tc_body,
        out_shape=jax.ShapeDtypeStruct((_B_TC, _F), jnp.float32),
        grid=(_B_TC // _TCB,),
        in_specs=[
            pl.BlockSpec((16, _F), lambda i: (0, 0)),
            pl.BlockSpec((_TCB, _F), lambda i: (i + off, 0)),
        ],
        out_specs=pl.BlockSpec((_TCB, _F), lambda i: (i, 0)),
    )(wt_host, x)


def _assemble_body(t_ref, full_ref, o_ref):
    del full_ref
    o_ref[...] = t_ref[...]


def _assemble(out_tc, full):
    off = _B_SC // _TCB
    return pl.pallas_call(
        _assemble_body,
        out_shape=jax.ShapeDtypeStruct((_BATCH, _F), jnp.float32),
        grid=(_B_TC // _TCB,),
        in_specs=[
            pl.BlockSpec((_TCB, _F), lambda i: (i, 0)),
            pl.BlockSpec(memory_space=pl.ANY),
        ],
        out_specs=pl.BlockSpec((_TCB, _F), lambda i: (i + off, 0)),
        input_output_aliases={1: 0},
    )(out_tc, full)


def kernel(x, w):
    wsub_host = lax.slice(w, (0, 16), (_F, 32))
    wt_host = wsub_host.T
    wflat_host = wsub_host.reshape(_F * 16 // 128, 128)
    full = _sc_half(x, wflat_host)
    out_tc = _tc_half(x, wt_host)
    return _assemble(out_tc, full)
